# baseline (speedup 1.0000x reference)
import jax
import jax.numpy as jnp
from jax.experimental import pallas as pl
from jax.experimental.pallas import tpu as pltpu

_BM = 6144


def _vq_logits_kernel(keys_ref, emb_ref, out_ref):
    keys = keys_ref[...]
    emb = emb_ref[...]
    bm = keys.shape[0]
    kk = emb.shape[0]
    f32 = jnp.float32
    bf16 = jnp.bfloat16

    k_sq = jnp.sum(keys * keys, axis=1, keepdims=True)
    k_hi = k_sq.astype(bf16)
    k_lo = (k_sq - k_hi.astype(f32)).astype(bf16)
    ones_b = jnp.ones((bm, 1), bf16)
    a = jnp.concatenate(
        [(keys + keys).astype(bf16), k_hi, k_lo, ones_b, ones_b], axis=1)

    e_sq = jnp.sum(emb * emb, axis=1, keepdims=True)
    e_hi = e_sq.astype(bf16)
    e_lo = (e_sq - e_hi.astype(f32)).astype(bf16)
    neg_ones_k = jnp.full((kk, 1), -1, bf16)
    b = jnp.concatenate(
        [emb.astype(bf16), neg_ones_k, neg_ones_k, -e_hi, -e_lo], axis=1)

    out_ref[...] = jax.lax.dot_general(
        a, b, (((1,), (1,)), ((), ())),
        preferred_element_type=f32)


def kernel(keys, embeddings):
    B, C = keys.shape
    K = embeddings.shape[0]
    return pl.pallas_call(
        _vq_logits_kernel,
        grid=(B // _BM,),
        in_specs=[
            pl.BlockSpec((_BM, C), lambda i: (i, 0)),
            pl.BlockSpec((K, C), lambda i: (0, 0)),
        ],
        out_specs=pl.BlockSpec((_BM, K), lambda i: (i, 0)),
        out_shape=jax.ShapeDtypeStruct((B, K), jnp.float32),
        compiler_params=pltpu.CompilerParams(
            dimension_semantics=("parallel",)),
    )(keys, embeddings)


# speedup vs baseline: 1.0508x; 1.0508x over previous
import jax
import jax.numpy as jnp
from jax.experimental import pallas as pl
from jax.experimental.pallas import tpu as pltpu

_BM = 2048


def _vq_logits_kernel(keys_ref, emb_ref, out_ref):
    keys = keys_ref[...]
    emb = emb_ref[...]
    bm = keys.shape[0]
    kk = emb.shape[0]
    f32 = jnp.float32
    bf16 = jnp.bfloat16

    k_sq = jnp.sum(keys * keys, axis=1, keepdims=True)
    k_hi = k_sq.astype(bf16)
    k_lo = (k_sq - k_hi.astype(f32)).astype(bf16)
    ones_b = jnp.ones((bm, 1), bf16)
    a = jnp.concatenate(
        [(keys + keys).astype(bf16), k_hi, k_lo, ones_b, ones_b], axis=1)

    e_sq = jnp.sum(emb * emb, axis=1, keepdims=True)
    e_hi = e_sq.astype(bf16)
    e_lo = (e_sq - e_hi.astype(f32)).astype(bf16)
    neg_ones_k = jnp.full((kk, 1), -1, bf16)
    b = jnp.concatenate(
        [emb.astype(bf16), neg_ones_k, neg_ones_k, -e_hi, -e_lo], axis=1)

    out_ref[...] = jax.lax.dot_general(
        a, b, (((1,), (1,)), ((), ())),
        preferred_element_type=f32)


def kernel(keys, embeddings):
    B, C = keys.shape
    K = embeddings.shape[0]
    return pl.pallas_call(
        _vq_logits_kernel,
        grid=(B // _BM,),
        in_specs=[
            pl.BlockSpec((_BM, C), lambda i: (i, 0)),
            pl.BlockSpec((K, C), lambda i: (0, 0)),
        ],
        out_specs=pl.BlockSpec((_BM, K), lambda i: (i, 0)),
        out_shape=jax.ShapeDtypeStruct((B, K), jnp.float32),
        compiler_params=pltpu.CompilerParams(
            dimension_semantics=("parallel",)),
    )(keys, embeddings)


# speedup vs baseline: 1.0639x; 1.0125x over previous
import jax
import jax.numpy as jnp
from jax.experimental import pallas as pl
from jax.experimental.pallas import tpu as pltpu

_BM = 2048


def _vq_logits_kernel(keys_ref, emb_ref, out_ref):
    keys = keys_ref[...]
    emb = emb_ref[...]
    bm = keys.shape[0]
    kk = emb.shape[0]
    f32 = jnp.float32
    bf16 = jnp.bfloat16

    k_sq = jnp.sum(keys * keys, axis=1, keepdims=True)
    k_hi = k_sq.astype(bf16)
    k_lo = (k_sq - k_hi.astype(f32)).astype(bf16)
    ones_b = jnp.ones((bm, 1), bf16)
    a = jnp.concatenate(
        [(keys + keys).astype(bf16), k_hi, k_lo, ones_b, ones_b], axis=1)

    e_sq = jnp.sum(emb * emb, axis=1, keepdims=True)
    e_hi = e_sq.astype(bf16)
    e_lo = (e_sq - e_hi.astype(f32)).astype(bf16)
    neg_ones_k = jnp.full((kk, 1), -1, bf16)
    b = jnp.concatenate(
        [emb.astype(bf16), neg_ones_k, neg_ones_k, -e_hi, -e_lo], axis=1)

    out_ref[...] = jax.lax.dot_general(
        a, b, (((1,), (1,)), ((), ())),
        preferred_element_type=f32)


def kernel(keys, embeddings):
    B, C = keys.shape
    K = embeddings.shape[0]
    return pl.pallas_call(
        _vq_logits_kernel,
        grid=(B // _BM,),
        in_specs=[
            pl.BlockSpec((_BM, C), lambda i: (i, 0)),
            pl.BlockSpec((K, C), lambda i: (0, 0)),
        ],
        out_specs=pl.BlockSpec((_BM, K), lambda i: (i, 0)),
        out_shape=jax.ShapeDtypeStruct((B, K), jnp.float32),
        compiler_params=pltpu.CompilerParams(
            dimension_semantics=("arbitrary",)),
    )(keys, embeddings)
